# baseline (speedup 1.0000x reference)
import functools

import jax
import jax.numpy as jnp
from jax.experimental import pallas as pl
from jax.experimental.pallas import tpu as pltpu

_CHUNK = 8192
_NEG_INF = float("-inf")

_ROT = ((13, 15, 26, 6), (17, 29, 16, 24))


def _rotl(x, r):
    return (x << jnp.uint32(r)) | (x >> jnp.uint32(32 - r))


def _threefry_bits(ctr):
    k0 = jnp.uint32(0)
    k1 = jnp.uint32(42)
    k2 = jnp.uint32(0x1BD11BDA) ^ k0 ^ k1
    ks = (k0, k1, k2)
    x0 = jnp.zeros_like(ctr) + k0
    x1 = ctr + k1
    for g in range(5):
        rot = _ROT[g % 2]
        for j in range(4):
            x0 = x0 + x1
            x1 = _rotl(x1, rot[j])
            x1 = x1 ^ x0
        x0 = x0 + ks[(g + 1) % 3]
        x1 = x1 + ks[(g + 2) % 3] + jnp.uint32(g + 1)
    return x0 ^ x1


def _gumbel_from_ctr(ctr):
    bits = _threefry_bits(ctr)
    mant = (bits >> jnp.uint32(9)) | jnp.uint32(0x3F800000)
    u = pltpu.bitcast(mant, jnp.float32) - jnp.float32(1.0)
    u = jnp.maximum(u + jnp.float32(1e-20), jnp.float32(1e-20))
    return -jnp.log(-jnp.log(u))


def _pass_a_body(nblk, vocab, l_ref, mp_ref, tok_ref, m_scr, s_scr, v_scr, i_scr):
    v = pl.program_id(0)
    rows, chunk = l_ref.shape

    @pl.when(v == 0)
    def _init():
        m_scr[...] = jnp.full_like(m_scr, _NEG_INF)
        s_scr[...] = jnp.zeros_like(s_scr)
        v_scr[...] = jnp.full_like(v_scr, _NEG_INF)
        i_scr[...] = jnp.zeros_like(i_scr)

    l = l_ref[...]
    col = jax.lax.broadcasted_iota(jnp.int32, (rows, chunk), 1) + v * chunk
    valid = col < vocab
    lm = jnp.where(valid, l, _NEG_INF)

    row = jax.lax.broadcasted_iota(jnp.int32, (rows, chunk), 0)
    ctr = pltpu.bitcast(row * vocab + col, jnp.uint32)
    g = _gumbel_from_ctr(ctr)
    score = lm + g

    bm = jnp.max(lm, axis=1, keepdims=True)
    m_old = m_scr[...]
    m_new = jnp.maximum(m_old, bm)
    s_scr[...] = s_scr[...] * jnp.exp(m_old - m_new) + jnp.sum(
        jnp.exp(lm - m_new), axis=1, keepdims=True
    )
    m_scr[...] = m_new

    bv = jnp.max(score, axis=1, keepdims=True)
    bi = jnp.min(
        jnp.where(score == bv, col, jnp.int32(2**31 - 1)), axis=1, keepdims=True
    )
    upd = bv > v_scr[...]
    v_scr[...] = jnp.where(upd, bv, v_scr[...])
    i_scr[...] = jnp.where(upd, bi, i_scr[...])

    @pl.when(v == nblk - 1)
    def _fin():
        mp_ref[...] = m_scr[...] + jnp.log(s_scr[...])
        tok_ref[...] = i_scr[...]


def _pass_b_body(l_ref, mp_ref, out_ref):
    out_ref[...] = jnp.exp(l_ref[...] - mp_ref[...])


def kernel(logits):
    rows, vocab = logits.shape
    chunk = min(_CHUNK, vocab)
    nblk = pl.cdiv(vocab, chunk)

    mp, tok = pl.pallas_call(
        functools.partial(_pass_a_body, nblk, vocab),
        grid=(nblk,),
        in_specs=[pl.BlockSpec((rows, chunk), lambda v: (0, v))],
        out_specs=[
            pl.BlockSpec((rows, 1), lambda v: (0, 0)),
            pl.BlockSpec((rows, 1), lambda v: (0, 0)),
        ],
        out_shape=[
            jax.ShapeDtypeStruct((rows, 1), jnp.float32),
            jax.ShapeDtypeStruct((rows, 1), jnp.int32),
        ],
        scratch_shapes=[
            pltpu.VMEM((rows, 1), jnp.float32),
            pltpu.VMEM((rows, 1), jnp.float32),
            pltpu.VMEM((rows, 1), jnp.float32),
            pltpu.VMEM((rows, 1), jnp.int32),
        ],
        compiler_params=pltpu.CompilerParams(
            dimension_semantics=("arbitrary",),
        ),
    )(logits)

    probs = pl.pallas_call(
        _pass_b_body,
        grid=(nblk,),
        in_specs=[
            pl.BlockSpec((rows, chunk), lambda v: (0, v)),
            pl.BlockSpec((rows, 1), lambda v: (0, 0)),
        ],
        out_specs=pl.BlockSpec((rows, chunk), lambda v: (0, v)),
        out_shape=jax.ShapeDtypeStruct((rows, vocab), jnp.float32),
        compiler_params=pltpu.CompilerParams(
            dimension_semantics=("arbitrary",),
        ),
    )(logits, mp)

    return probs, tok.reshape(rows)
